# Initial kernel scaffold; baseline (speedup 1.0000x reference)
import jax
import jax.numpy as jnp
from jax.experimental import pallas as pl


def kernel(xyz, color):
    raise NotImplementedError("write your pallas kernel here")


# speedup vs baseline: 1.5891x; 1.5891x over previous
import functools

import jax
import jax.numpy as jnp
from jax.experimental import pallas as pl
from jax.experimental.pallas import tpu as pltpu

B, N, G, K = 8, 8192, 512, 32
R, C = 8, 1024


def _fps_body(px_ref, py_ref, pz_ref, center_ref):
    px = px_ref[...]
    py = py_ref[...]
    pz = pz_ref[...]
    ii = (jax.lax.broadcasted_iota(jnp.int32, (B, R, C), 1) * C
          + jax.lax.broadcasted_iota(jnp.int32, (B, R, C), 2))

    lane = jax.lax.broadcasted_iota(jnp.int32, (1, 1, 3), 2)
    e0 = (lane == 0).astype(jnp.float32)
    e1 = (lane == 1).astype(jnp.float32)
    e2 = (lane == 2).astype(jnp.float32)

    lx0 = px[:, 0:1, 0:1]
    ly0 = py[:, 0:1, 0:1]
    lz0 = pz[:, 0:1, 0:1]
    center_ref[:, 0:1, :] = lx0 * e0 + ly0 * e1 + lz0 * e2

    def body(i, carry):
        dists, lx, ly, lz = carry
        dx = px - lx
        dy = py - ly
        dz = pz - lz
        d = (dx * dx + dy * dy) + dz * dz
        dists = jnp.minimum(dists, d)
        m = jnp.max(dists, axis=(1, 2), keepdims=True)
        cand = jnp.where(dists == m, ii, N)
        nxt = jnp.min(cand, axis=(1, 2), keepdims=True)
        onehot = ii == nxt
        zero = jnp.zeros((), jnp.float32)
        nlx = jnp.sum(jnp.where(onehot, px, zero), axis=(1, 2), keepdims=True)
        nly = jnp.sum(jnp.where(onehot, py, zero), axis=(1, 2), keepdims=True)
        nlz = jnp.sum(jnp.where(onehot, pz, zero), axis=(1, 2), keepdims=True)
        center_ref[:, pl.ds(i, 1), :] = nlx * e0 + nly * e1 + nlz * e2
        return dists, nlx, nly, nlz

    init = (jnp.full((B, R, C), jnp.inf, jnp.float32), lx0, ly0, lz0)
    jax.lax.fori_loop(1, G, body, init)


def _fps_centers(xyz):
    px = xyz[:, :, 0].reshape(B, R, C)
    py = xyz[:, :, 1].reshape(B, R, C)
    pz = xyz[:, :, 2].reshape(B, R, C)
    return pl.pallas_call(
        _fps_body,
        out_shape=jax.ShapeDtypeStruct((B, G, 3), jnp.float32),
    )(px, py, pz)


def kernel(xyz, color):
    center = _fps_centers(xyz)
    dist = -2.0 * jnp.matmul(center, jnp.swapaxes(xyz, 1, 2))
    dist = dist + jnp.sum(center ** 2, axis=-1)[:, :, None]
    dist = dist + jnp.sum(xyz ** 2, axis=-1)[:, None, :]
    _, idx = jax.lax.top_k(-dist, K)
    gather = jax.vmap(lambda pts, i: pts[i])
    neighborhood = gather(xyz, idx)
    neighborhood_color = gather(color, idx)
    neighborhood = neighborhood - center[:, :, None, :]
    features = jnp.concatenate([neighborhood, neighborhood_color], axis=-1)
    return (neighborhood, center, features)


# speedup vs baseline: 2.2566x; 1.4200x over previous
import functools

import jax
import jax.numpy as jnp
from jax.experimental import pallas as pl
from jax.experimental.pallas import tpu as pltpu

B, N, G, K = 8, 8192, 512, 32
R, C = 8, 1024


def _fps_body(px_ref, py_ref, pz_ref, center_ref):
    px = px_ref[...]
    py = py_ref[...]
    pz = pz_ref[...]
    ii = (jax.lax.broadcasted_iota(jnp.int32, (B, R, C), 1) * C
          + jax.lax.broadcasted_iota(jnp.int32, (B, R, C), 2))

    lane = jax.lax.broadcasted_iota(jnp.int32, (1, 1, 3), 2)
    e0 = (lane == 0).astype(jnp.float32)
    e1 = (lane == 1).astype(jnp.float32)
    e2 = (lane == 2).astype(jnp.float32)

    lx0 = px[:, 0:1, 0:1]
    ly0 = py[:, 0:1, 0:1]
    lz0 = pz[:, 0:1, 0:1]
    center_ref[:, 0:1, :] = lx0 * e0 + ly0 * e1 + lz0 * e2

    def body(i, carry):
        dists, lx, ly, lz = carry
        dx = px - lx
        dy = py - ly
        dz = pz - lz
        d = (dx * dx + dy * dy) + dz * dz
        dists = jnp.minimum(dists, d)
        m = jnp.max(dists, axis=(1, 2), keepdims=True)
        cand = jnp.where(dists == m, ii, N)
        nxt = jnp.min(cand, axis=(1, 2), keepdims=True)
        onehot = ii == nxt
        zero = jnp.zeros((), jnp.float32)
        nlx = jnp.sum(jnp.where(onehot, px, zero), axis=(1, 2), keepdims=True)
        nly = jnp.sum(jnp.where(onehot, py, zero), axis=(1, 2), keepdims=True)
        nlz = jnp.sum(jnp.where(onehot, pz, zero), axis=(1, 2), keepdims=True)
        center_ref[:, pl.ds(i, 1), :] = nlx * e0 + nly * e1 + nlz * e2
        return dists, nlx, nly, nlz

    init = (jnp.full((B, R, C), jnp.inf, jnp.float32), lx0, ly0, lz0)
    jax.lax.fori_loop(1, G, body, init)


def _fps_centers(xyz):
    px = xyz[:, :, 0].reshape(B, R, C)
    py = xyz[:, :, 1].reshape(B, R, C)
    pz = xyz[:, :, 2].reshape(B, R, C)
    return pl.pallas_call(
        _fps_body,
        out_shape=jax.ShapeDtypeStruct((B, G, 3), jnp.float32),
    )(px, py, pz)


GT = 8


def _topk_body(cen_ref, xt_ref, idx_ref):
    xt = xt_ref[0]
    cen = cen_ref[0]
    mm = jax.lax.dot_general(
        cen.astype(jnp.bfloat16), xt.astype(jnp.bfloat16),
        (((1,), (0,)), ((), ())),
        preferred_element_type=jnp.float32)
    cx = cen[:, 0:1]
    cy = cen[:, 1:2]
    cz = cen[:, 2:3]
    s1 = (cx * cx + cy * cy) + cz * cz
    px = xt[0:1, :]
    py = xt[1:2, :]
    pz = xt[2:3, :]
    s2 = (px * px + py * py) + pz * pz
    d = ((-2.0 * mm) + s1) + s2
    ii = jax.lax.broadcasted_iota(jnp.int32, (GT, N), 1)
    inf = jnp.float32(jnp.inf)
    for j in range(K):
        m = jnp.min(d, axis=1, keepdims=True)
        cand = jnp.where(d == m, ii, N)
        idx = jnp.min(cand, axis=1, keepdims=True)
        idx_ref[0, :, j:j + 1] = idx
        d = jnp.where(ii == idx, inf, d)


def _knn_idx(center, xt):
    return pl.pallas_call(
        _topk_body,
        grid=(B, G // GT),
        in_specs=[
            pl.BlockSpec((1, GT, 3), lambda b, g: (b, g, 0)),
            pl.BlockSpec((1, 3, N), lambda b, g: (b, 0, 0)),
        ],
        out_specs=pl.BlockSpec((1, GT, K), lambda b, g: (b, g, 0)),
        out_shape=jax.ShapeDtypeStruct((B, G, K), jnp.int32),
    )(center, xt)


def kernel(xyz, color):
    center = _fps_centers(xyz)
    idx = _knn_idx(center, jnp.swapaxes(xyz, 1, 2))
    gather = jax.vmap(lambda pts, i: pts[i])
    neighborhood = gather(xyz, idx)
    neighborhood_color = gather(color, idx)
    neighborhood = neighborhood - center[:, :, None, :]
    features = jnp.concatenate([neighborhood, neighborhood_color], axis=-1)
    return (neighborhood, center, features)


# speedup vs baseline: 3.9762x; 1.7620x over previous
import functools

import jax
import jax.numpy as jnp
from jax.experimental import pallas as pl
from jax.experimental.pallas import tpu as pltpu

B, N, G, K = 8, 8192, 512, 32
R, C = 8, 1024


def _fps_body(px_ref, py_ref, pz_ref, center_ref):
    px = px_ref[...]
    py = py_ref[...]
    pz = pz_ref[...]
    ii = (jax.lax.broadcasted_iota(jnp.int32, (B, R, C), 1) * C
          + jax.lax.broadcasted_iota(jnp.int32, (B, R, C), 2))

    lane = jax.lax.broadcasted_iota(jnp.int32, (1, 1, 3), 2)
    e0 = (lane == 0).astype(jnp.float32)
    e1 = (lane == 1).astype(jnp.float32)
    e2 = (lane == 2).astype(jnp.float32)

    lx0 = px[:, 0:1, 0:1]
    ly0 = py[:, 0:1, 0:1]
    lz0 = pz[:, 0:1, 0:1]
    center_ref[:, 0:1, :] = lx0 * e0 + ly0 * e1 + lz0 * e2

    def body(i, carry):
        dists, lx, ly, lz = carry
        dx = px - lx
        dy = py - ly
        dz = pz - lz
        d = (dx * dx + dy * dy) + dz * dz
        dists = jnp.minimum(dists, d)
        m = jnp.max(dists, axis=(1, 2), keepdims=True)
        cand = jnp.where(dists == m, ii, N)
        nxt = jnp.min(cand, axis=(1, 2), keepdims=True)
        onehot = ii == nxt
        zero = jnp.zeros((), jnp.float32)
        nlx = jnp.sum(jnp.where(onehot, px, zero), axis=(1, 2), keepdims=True)
        nly = jnp.sum(jnp.where(onehot, py, zero), axis=(1, 2), keepdims=True)
        nlz = jnp.sum(jnp.where(onehot, pz, zero), axis=(1, 2), keepdims=True)
        center_ref[:, pl.ds(i, 1), :] = nlx * e0 + nly * e1 + nlz * e2
        return dists, nlx, nly, nlz

    init = (jnp.full((B, R, C), jnp.inf, jnp.float32), lx0, ly0, lz0)
    jax.lax.fori_loop(1, G, body, init)


def _fps_centers(xyz):
    px = xyz[:, :, 0].reshape(B, R, C)
    py = xyz[:, :, 1].reshape(B, R, C)
    pz = xyz[:, :, 2].reshape(B, R, C)
    return pl.pallas_call(
        _fps_body,
        out_shape=jax.ShapeDtypeStruct((B, G, 3), jnp.float32),
    )(px, py, pz)


GT = 8


def _topk_body(cen_ref, xt_ref, idx_ref):
    xt = xt_ref[0]
    cen = cen_ref[0]
    mm = jax.lax.dot_general(
        cen.astype(jnp.bfloat16), xt.astype(jnp.bfloat16),
        (((1,), (0,)), ((), ())),
        preferred_element_type=jnp.float32)
    cx = cen[:, 0:1]
    cy = cen[:, 1:2]
    cz = cen[:, 2:3]
    s1 = (cx * cx + cy * cy) + cz * cz
    px = xt[0:1, :]
    py = xt[1:2, :]
    pz = xt[2:3, :]
    s2 = (px * px + py * py) + pz * pz
    d = ((-2.0 * mm) + s1) + s2
    ii = jax.lax.broadcasted_iota(jnp.int32, (GT, N), 1)
    inf = jnp.float32(jnp.inf)
    for j in range(K):
        m = jnp.min(d, axis=1, keepdims=True)
        cand = jnp.where(d == m, ii, N)
        idx = jnp.min(cand, axis=1, keepdims=True)
        idx_ref[0, :, j:j + 1] = idx
        d = jnp.where(ii == idx, inf, d)


def _knn_idx(center, xt):
    return pl.pallas_call(
        _topk_body,
        grid=(B, G // GT),
        in_specs=[
            pl.BlockSpec((1, GT, 3), lambda b, g: (b, g, 0)),
            pl.BlockSpec((1, 3, N), lambda b, g: (b, 0, 0)),
        ],
        out_specs=pl.BlockSpec((1, GT, K), lambda b, g: (b, g, 0)),
        out_shape=jax.ShapeDtypeStruct((B, G, K), jnp.int32),
    )(center, xt)


NW = 32
GPW = (B * G) // NW
BPW = NW // B


def _gather_body(xyz_hbm, col_hbm, idx_hbm, cen_hbm, nb_hbm, ft_hbm,
                 xyzv, colv, idxv, cenv, nbv, ftv, sem):
    from jax.experimental.pallas import tpu_sc as plsc
    w = jax.lax.axis_index("s") * 2 + jax.lax.axis_index("c")
    b = w // BPW
    g0 = (w % BPW) * GPW

    pltpu.sync_copy(xyz_hbm.at[pl.ds(b * (N * 3), N * 3)], xyzv)
    pltpu.sync_copy(col_hbm.at[pl.ds(b * (N * 3), N * 3)], colv)
    pltpu.sync_copy(idx_hbm.at[pl.ds((b * G + g0) * K, GPW * K)], idxv)
    pltpu.sync_copy(cen_hbm.at[pl.ds((b * G + g0) * 3, GPW * 3)], cenv)

    lane = jax.lax.iota(jnp.int32, 16)
    consts = []
    for j in range(6):
        q = lane + (j * 16)
        d3 = jax.lax.shift_right_logical(q * 21846, 16)
        m3 = q - d3 * 3
        consts.append((d3, m3))

    def body(g, _):
        for j in range(6):
            d3, m3 = consts[j]
            pt = plsc.load_gather(idxv, [g * K + d3])
            elem = pt * 3 + m3
            xv = plsc.load_gather(xyzv, [elem])
            cv = plsc.load_gather(cenv, [g * 3 + m3])
            nb = xv - cv
            nbv[pl.ds(g * 96 + j * 16, 16)] = nb
            fq = g * 192 + d3 * 6 + m3
            plsc.store_scatter(ftv, [fq], nb)
            cl = plsc.load_gather(colv, [elem])
            plsc.store_scatter(ftv, [fq + 3], cl)
        return 0

    jax.lax.fori_loop(0, GPW, body, 0)

    pltpu.sync_copy(nbv, nb_hbm.at[pl.ds((b * G + g0) * 96, GPW * 96)])
    pltpu.sync_copy(ftv, ft_hbm.at[pl.ds((b * G + g0) * 192, GPW * 192)])


def _sc_gather(xyz, color, idx, center):
    from jax.experimental.pallas import tpu_sc as plsc
    mesh = plsc.VectorSubcoreMesh(core_axis_name="c", subcore_axis_name="s")
    f = functools.partial(
        pl.kernel,
        out_type=[
            jax.ShapeDtypeStruct((B * G * K * 3,), jnp.float32),
            jax.ShapeDtypeStruct((B * G * K * 6,), jnp.float32),
        ],
        mesh=mesh,
        compiler_params=pltpu.CompilerParams(needs_layout_passes=False),
        scratch_types=[
            pltpu.VMEM((N * 3,), jnp.float32),
            pltpu.VMEM((N * 3,), jnp.float32),
            pltpu.VMEM((GPW * K,), jnp.int32),
            pltpu.VMEM((GPW * 3,), jnp.float32),
            pltpu.VMEM((GPW * 96,), jnp.float32),
            pltpu.VMEM((GPW * 192,), jnp.float32),
            pltpu.SemaphoreType.DMA,
        ],
    )(_gather_body)
    nb, ft = f(xyz.reshape(-1), color.reshape(-1), idx.reshape(-1),
               center.reshape(-1))
    return nb.reshape(B, G, K, 3), ft.reshape(B, G, K, 6)


def kernel(xyz, color):
    center = _fps_centers(xyz)
    idx = _knn_idx(center, jnp.swapaxes(xyz, 1, 2))
    neighborhood, features = _sc_gather(xyz, color, idx, center)
    return (neighborhood, center, features)


# speedup vs baseline: 12.0848x; 3.0393x over previous
import functools

import jax
import jax.numpy as jnp
from jax.experimental import pallas as pl
from jax.experimental.pallas import tpu as pltpu

B, N, G, K = 8, 8192, 512, 32
R, C = 8, 1024


def _fps_body(px_ref, py_ref, pz_ref, center_ref):
    px = px_ref[...]
    py = py_ref[...]
    pz = pz_ref[...]
    ii = (jax.lax.broadcasted_iota(jnp.int32, (B, R, C), 1) * C
          + jax.lax.broadcasted_iota(jnp.int32, (B, R, C), 2))

    lane = jax.lax.broadcasted_iota(jnp.int32, (1, 1, 3), 2)
    e0 = (lane == 0).astype(jnp.float32)
    e1 = (lane == 1).astype(jnp.float32)
    e2 = (lane == 2).astype(jnp.float32)

    lx0 = px[:, 0:1, 0:1]
    ly0 = py[:, 0:1, 0:1]
    lz0 = pz[:, 0:1, 0:1]
    center_ref[:, 0:1, :] = lx0 * e0 + ly0 * e1 + lz0 * e2

    def body(i, carry):
        dists, lx, ly, lz = carry
        dx = px - lx
        dy = py - ly
        dz = pz - lz
        d = (dx * dx + dy * dy) + dz * dz
        dists = jnp.minimum(dists, d)
        m = jnp.max(dists, axis=(1, 2), keepdims=True)
        cand = jnp.where(dists == m, ii, N)
        nxt = jnp.min(cand, axis=(1, 2), keepdims=True)
        onehot = ii == nxt
        zero = jnp.zeros((), jnp.float32)
        nlx = jnp.sum(jnp.where(onehot, px, zero), axis=(1, 2), keepdims=True)
        nly = jnp.sum(jnp.where(onehot, py, zero), axis=(1, 2), keepdims=True)
        nlz = jnp.sum(jnp.where(onehot, pz, zero), axis=(1, 2), keepdims=True)
        center_ref[:, pl.ds(i, 1), :] = nlx * e0 + nly * e1 + nlz * e2
        return dists, nlx, nly, nlz

    init = (jnp.full((B, R, C), jnp.inf, jnp.float32), lx0, ly0, lz0)
    jax.lax.fori_loop(1, G, body, init)


def _fps_centers(xyz):
    px = xyz[:, :, 0].reshape(B, R, C)
    py = xyz[:, :, 1].reshape(B, R, C)
    pz = xyz[:, :, 2].reshape(B, R, C)
    return pl.pallas_call(
        _fps_body,
        out_shape=jax.ShapeDtypeStruct((B, G, 3), jnp.float32),
    )(px, py, pz)


GT = 8


def _dist_body(cen_ref, xt_ref, d_ref, t_ref):
    xt = xt_ref[0]
    cen = cen_ref[0]
    mm = jax.lax.dot_general(
        cen.astype(jnp.bfloat16), xt.astype(jnp.bfloat16),
        (((1,), (0,)), ((), ())),
        preferred_element_type=jnp.float32)
    cx = cen[:, 0:1]
    cy = cen[:, 1:2]
    cz = cen[:, 2:3]
    s1 = (cx * cx + cy * cy) + cz * cz
    px = xt[0:1, :]
    py = xt[1:2, :]
    pz = xt[2:3, :]
    s2 = (px * px + py * py) + pz * pz
    d = ((-2.0 * mm) + s1) + s2
    d_ref[0] = d
    cm = jnp.min(d.reshape(GT, 32, N // 32), axis=2)
    t_ref[0] = jnp.max(cm, axis=1, keepdims=True)


def _dist_thresh(center, xt):
    return pl.pallas_call(
        _dist_body,
        grid=(B, G // GT),
        in_specs=[
            pl.BlockSpec((1, GT, 3), lambda b, g: (b, g, 0)),
            pl.BlockSpec((1, 3, N), lambda b, g: (b, 0, 0)),
        ],
        out_specs=[
            pl.BlockSpec((1, GT, N), lambda b, g: (b, g, 0)),
            pl.BlockSpec((1, GT, 1), lambda b, g: (b, g, 0)),
        ],
        out_shape=[
            jax.ShapeDtypeStruct((B, G, N), jnp.float32),
            jax.ShapeDtypeStruct((B, G, 1), jnp.float32),
        ],
    )(center, xt)


NW = 32
GPW = (B * G) // NW
BPW = NW // B


def _sc_body(xyz_hbm, col_hbm, cen_hbm, d_hbm, t_hbm, nb_hbm, ft_hbm,
             xyzv, colv, cenv, tv, dbuf, idxbuf, idx32, nbv, ftv):
    from jax.experimental.pallas import tpu_sc as plsc
    w = jax.lax.axis_index("s") * 2 + jax.lax.axis_index("c")
    b = w // BPW
    g0 = (w % BPW) * GPW
    row0 = b * G + g0

    pltpu.sync_copy(xyz_hbm.at[pl.ds(b * (N * 3), N * 3)], xyzv)
    pltpu.sync_copy(col_hbm.at[pl.ds(b * (N * 3), N * 3)], colv)
    pltpu.sync_copy(cen_hbm.at[pl.ds(row0 * 3, GPW * 3)], cenv)
    pltpu.sync_copy(t_hbm.at[pl.ds(row0, GPW)], tv)

    lane = jax.lax.iota(jnp.int32, 16)
    inf16 = (lane * 0).astype(jnp.float32) + jnp.float32(jnp.inf)
    dbuf[pl.ds(N, 16)] = inf16
    consts = []
    for j in range(6):
        q = lane + (j * 16)
        d3 = jax.lax.shift_right_logical(q * 21846, 16)
        m3 = q - d3 * 3
        consts.append((d3, m3))

    def rev(x):
        return jax.lax.rev(x, (0,))

    def sort16(k, v):
        return plsc.sort_key_val(k, v)

    def merge16(ak, av, bk, bv):
        rbk = rev(bk)
        rbv = rev(bv)
        s = ak <= rbk
        mk0 = jnp.minimum(ak, rbk)
        mv0 = jnp.where(s, av, rbv)
        mk1 = jnp.maximum(ak, rbk)
        mv1 = jnp.where(s, rbv, av)
        mk0, mv0 = sort16(mk0, mv0)
        mk1, mv1 = sort16(mk1, mv1)
        return mk0, mv0, mk1, mv1

    def load_sorted32(off):
        i0 = idxbuf[pl.ds(off, 16)]
        i1 = idxbuf[pl.ds(off + 16, 16)]
        k0 = plsc.load_gather(dbuf, [i0])
        k1 = plsc.load_gather(dbuf, [i1])
        k0, i0 = sort16(k0, i0)
        k1, i1 = sort16(k1, i1)
        return merge16(k0, i0, k1, i1)

    def row_body(r, _):
        pltpu.sync_copy(d_hbm.at[pl.ds((row0 + r) * N, N)], dbuf.at[pl.ds(0, N)])
        t16 = plsc.load_gather(tv, [lane * 0 + r])

        def scan_body(i, off):
            base = lane + i * 16
            d16 = dbuf[pl.ds(i * 16, 16)]
            msk = d16 <= t16
            plsc.store_compressed(idxbuf.at[pl.ds(off, 16)], base, mask=msk)
            cnt = plsc.all_reduce_population_count(msk)
            return off + cnt[0]

        m_cnt = jax.lax.fori_loop(0, N // 16, scan_body, 0)
        sent = lane * 0 + N
        idxbuf[pl.ds(m_cnt, 16)] = sent
        idxbuf[pl.ds(m_cnt + 16, 16)] = sent

        acc = load_sorted32(0)
        nblk = jax.lax.shift_right_logical(m_cnt + 31, 5)

        def sel_body(blk, acc):
            a0k, a0v, a1k, a1v = acc
            n0k, n0v, n1k, n1v = load_sorted32(blk * 32)
            rk1 = rev(n1k)
            rv1 = rev(n1v)
            s0 = a0k <= rk1
            lo0k = jnp.minimum(a0k, rk1)
            lo0v = jnp.where(s0, a0v, rv1)
            rk0 = rev(n0k)
            rv0 = rev(n0v)
            s1 = a1k <= rk0
            lo1k = jnp.minimum(a1k, rk0)
            lo1v = jnp.where(s1, a1v, rv0)
            lo0k, lo0v = sort16(lo0k, lo0v)
            lo1k, lo1v = sort16(lo1k, lo1v)
            return merge16(lo0k, lo0v, lo1k, lo1v)

        acc = jax.lax.fori_loop(1, nblk, sel_body, acc)
        idx32[pl.ds(0, 16)] = acc[1]
        idx32[pl.ds(16, 16)] = acc[3]

        for j in range(6):
            d3, m3 = consts[j]
            pt = plsc.load_gather(idx32, [d3])
            elem = pt * 3 + m3
            xv = plsc.load_gather(xyzv, [elem])
            cv = plsc.load_gather(cenv, [r * 3 + m3])
            nb = xv - cv
            nbv[pl.ds(r * 96 + j * 16, 16)] = nb
            fq = r * 192 + d3 * 6 + m3
            plsc.store_scatter(ftv, [fq], nb)
            cl = plsc.load_gather(colv, [elem])
            plsc.store_scatter(ftv, [fq + 3], cl)
        return 0

    jax.lax.fori_loop(0, GPW, row_body, 0)

    pltpu.sync_copy(nbv, nb_hbm.at[pl.ds(row0 * 96, GPW * 96)])
    pltpu.sync_copy(ftv, ft_hbm.at[pl.ds(row0 * 192, GPW * 192)])


def _sc_select_gather(xyz, color, center, d, t):
    from jax.experimental.pallas import tpu_sc as plsc
    mesh = plsc.VectorSubcoreMesh(core_axis_name="c", subcore_axis_name="s")
    f = functools.partial(
        pl.kernel,
        out_type=[
            jax.ShapeDtypeStruct((B * G * K * 3,), jnp.float32),
            jax.ShapeDtypeStruct((B * G * K * 6,), jnp.float32),
        ],
        mesh=mesh,
        compiler_params=pltpu.CompilerParams(needs_layout_passes=False),
        scratch_types=[
            pltpu.VMEM((N * 3,), jnp.float32),
            pltpu.VMEM((N * 3,), jnp.float32),
            pltpu.VMEM((GPW * 3,), jnp.float32),
            pltpu.VMEM((GPW,), jnp.float32),
            pltpu.VMEM((N + 16,), jnp.float32),
            pltpu.VMEM((N + 48,), jnp.int32),
            pltpu.VMEM((32,), jnp.int32),
            pltpu.VMEM((GPW * 96,), jnp.float32),
            pltpu.VMEM((GPW * 192,), jnp.float32),
        ],
    )(_sc_body)
    nb, ft = f(xyz.reshape(-1), color.reshape(-1), center.reshape(-1),
               d.reshape(-1), t.reshape(-1))
    return nb.reshape(B, G, K, 3), ft.reshape(B, G, K, 6)


def kernel(xyz, color):
    center = _fps_centers(xyz)
    d, t = _dist_thresh(center, jnp.swapaxes(xyz, 1, 2))
    neighborhood, features = _sc_select_gather(xyz, color, center, d, t)
    return (neighborhood, center, features)
